# baseline (speedup 1.0000x reference)
import functools

import jax
import jax.numpy as jnp
from jax import lax
from jax.experimental import pallas as pl
from jax.experimental.pallas import tpu as pltpu
from jax.experimental.pallas import tpu_sc as plsc

N = 1048576
LANES = 16
BLK = 128
NUM_BLOCKS = N // BLK
VECS_PER_BLK = BLK // LANES

SC_BLOCKS = 4096
TC_BLOCKS = NUM_BLOCKS - SC_BLOCKS

NUM_WORKERS = 32
BLOCKS_PER_WORKER = SC_BLOCKS // NUM_WORKERS
ROWS_PER_WORKER = BLOCKS_PER_WORKER * BLK
NUM_CHUNKS = 4
BLOCKS_PER_CHUNK = BLOCKS_PER_WORKER // NUM_CHUNKS
ROWS_PER_CHUNK = ROWS_PER_WORKER // NUM_CHUNKS

TC_CHUNK = 512
TC_STEPS = TC_BLOCKS // TC_CHUNK


def _f1_counts_sc(xview, target):
  mesh = plsc.VectorSubcoreMesh(core_axis_name="c", subcore_axis_name="s")

  @functools.partial(
      pl.kernel,
      mesh=mesh,
      out_type=jax.ShapeDtypeStruct((NUM_WORKERS, LANES), jnp.int32),
      scratch_types=[
          pltpu.VMEM((BLOCKS_PER_WORKER, 2, BLK), jnp.float32),
          pltpu.VMEM((ROWS_PER_WORKER,), jnp.int32),
          pltpu.VMEM((LANES,), jnp.int32),
      ]
      + [pltpu.SemaphoreType.DMA] * (2 * NUM_CHUNKS),
      compiler_params=pltpu.CompilerParams(needs_layout_passes=False),
  )
  def body(x_hbm, t_hbm, out_hbm, x_v, t_v, stage_v, *sems):
    wid = lax.axis_index("s") * 2 + lax.axis_index("c")
    blk0 = wid * BLOCKS_PER_WORKER
    row0 = wid * ROWS_PER_WORKER

    copies = []
    for g in range(NUM_CHUNKS):
      cp_x = pltpu.async_copy(
          x_hbm.at[pl.ds(blk0 + g * BLOCKS_PER_CHUNK, BLOCKS_PER_CHUNK)],
          x_v.at[pl.ds(g * BLOCKS_PER_CHUNK, BLOCKS_PER_CHUNK)],
          sems[2 * g])
      cp_t = pltpu.async_copy(
          t_hbm.at[pl.ds(row0 + g * ROWS_PER_CHUNK, ROWS_PER_CHUNK)],
          t_v.at[pl.ds(g * ROWS_PER_CHUNK, ROWS_PER_CHUNK)],
          sems[2 * g + 1])
      copies.append((cp_x, cp_t))

    lane = lax.iota(jnp.int32, LANES)
    zero = jnp.zeros((LANES,), jnp.int32)
    one = jnp.ones((LANES,), jnp.int32)

    def blk_step(k, carry):
      acc_tp, acc_p, acc_t = carry
      for j in range(VECS_PER_BLK):
        c0 = x_v[k, 0, pl.ds(j * LANES, LANES)]
        c1 = x_v[k, 1, pl.ds(j * LANES, LANES)]
        tv = t_v[pl.ds(k * BLK + j * LANES, LANES)]
        pred = c1 > c0
        acc_p = acc_p + jnp.where(pred, one, zero)
        acc_tp = acc_tp + jnp.where(pred, tv, zero)
        acc_t = acc_t + tv
      return (acc_tp, acc_p, acc_t)

    acc = (zero, zero, zero)
    for g in range(NUM_CHUNKS):
      cp_x, cp_t = copies[g]
      cp_x.wait()
      cp_t.wait()
      acc = lax.fori_loop(
          g * BLOCKS_PER_CHUNK, (g + 1) * BLOCKS_PER_CHUNK, blk_step, acc)
    acc_tp, acc_p, acc_t = acc

    tp_s = jnp.sum(acc_tp)
    p_s = jnp.sum(acc_p)
    t_s = jnp.sum(acc_t)
    stage = jnp.where(lane == 0, tp_s,
                      jnp.where(lane == 1, p_s,
                                jnp.where(lane == 2, t_s, 0)))
    stage_v[...] = stage
    pltpu.sync_copy(stage_v, out_hbm.at[wid])

  return body(xview, target)


def _f1_counts_tc(xview_tc, tview_tc):
  def body(x_ref, t_ref, out_ref):
    i = pl.program_id(0)
    v = x_ref[...]
    q = pltpu.roll(v, 2 * TC_CHUNK - 1, 0)
    row = lax.broadcasted_iota(jnp.int32, (2 * TC_CHUNK, BLK), 0)
    even = (row & 1) == 0
    pred = (q > v) & even
    tv = t_ref[...]
    te = jnp.broadcast_to(tv[:, None, :], (TC_CHUNK, 2, BLK)).reshape(
        2 * TC_CHUNK, BLK)
    zero2 = jnp.zeros((2 * TC_CHUNK, BLK), jnp.int32)
    one2 = jnp.ones((2 * TC_CHUNK, BLK), jnp.int32)
    p_s = jnp.sum(jnp.where(pred, one2, zero2))
    tp_s = jnp.sum(jnp.where(pred, te, zero2))
    t_s = jnp.sum(tv)

    @pl.when(i == 0)
    def _init():
      out_ref[0, 0] = tp_s
      out_ref[0, 1] = p_s
      out_ref[0, 2] = t_s

    @pl.when(i != 0)
    def _acc():
      out_ref[0, 0] += tp_s
      out_ref[0, 1] += p_s
      out_ref[0, 2] += t_s

  return pl.pallas_call(
      body,
      grid=(TC_STEPS,),
      in_specs=[
          pl.BlockSpec((2 * TC_CHUNK, BLK),
                       lambda i: (SC_BLOCKS // TC_CHUNK + i, 0)),
          pl.BlockSpec((TC_CHUNK, BLK),
                       lambda i: (SC_BLOCKS // TC_CHUNK + i, 0)),
      ],
      out_specs=pl.BlockSpec(memory_space=pltpu.SMEM),
      out_shape=jax.ShapeDtypeStruct((1, 3), jnp.int32),
      compiler_params=pltpu.CompilerParams(
          dimension_semantics=("arbitrary",)),
  )(xview_tc, tview_tc)


def _finalize_tc(parts_sc, parts_tc):
  def fin(sc_ref, tc_ref, out_ref):
    x = sc_ref[...]
    col = lax.broadcasted_iota(jnp.int32, (NUM_WORKERS, LANES), 1)
    zero = jnp.zeros((NUM_WORKERS, LANES), jnp.int32)
    tp = (jnp.sum(jnp.where(col == 0, x, zero)) + tc_ref[0, 0]).astype(
        jnp.float32)
    p = (jnp.sum(jnp.where(col == 1, x, zero)) + tc_ref[0, 1]).astype(
        jnp.float32)
    t = (jnp.sum(jnp.where(col == 2, x, zero)) + tc_ref[0, 2]).astype(
        jnp.float32)
    fp = p - tp
    fn = t - tp
    precision = tp / (tp + fp + 1e-10)
    recall = tp / (tp + fn + 1e-10)
    out_ref[0, 0] = 2 * precision * recall / (precision + recall + 1e-10)

  out = pl.pallas_call(
      fin,
      in_specs=[
          pl.BlockSpec(memory_space=pltpu.VMEM),
          pl.BlockSpec(memory_space=pltpu.SMEM),
      ],
      out_specs=pl.BlockSpec(memory_space=pltpu.SMEM),
      out_shape=jax.ShapeDtypeStruct((1, 1), jnp.float32),
  )(parts_sc, parts_tc)
  return out.reshape(())


@jax.jit
def kernel(output, target):
  xview = output.reshape(NUM_BLOCKS, BLK, 2).transpose(0, 2, 1)
  tview = target.reshape(NUM_BLOCKS, BLK)
  parts_sc = _f1_counts_sc(xview, target)
  parts_tc = _f1_counts_tc(xview.reshape(2 * NUM_BLOCKS, BLK), tview)
  return _finalize_tc(parts_sc, parts_tc)


# speedup vs baseline: 1.0205x; 1.0205x over previous
import functools

import jax
import jax.numpy as jnp
from jax import lax
from jax.experimental import pallas as pl
from jax.experimental.pallas import tpu as pltpu
from jax.experimental.pallas import tpu_sc as plsc

N = 1048576
LANES = 16
BLK = 128
NUM_BLOCKS = N // BLK
VECS_PER_BLK = BLK // LANES

SC_BLOCKS = 4608
TC_BLOCKS = NUM_BLOCKS - SC_BLOCKS

NUM_WORKERS = 32
BLOCKS_PER_WORKER = SC_BLOCKS // NUM_WORKERS
ROWS_PER_WORKER = BLOCKS_PER_WORKER * BLK
NUM_CHUNKS = 6
BLOCKS_PER_CHUNK = BLOCKS_PER_WORKER // NUM_CHUNKS
ROWS_PER_CHUNK = ROWS_PER_WORKER // NUM_CHUNKS

TC_CHUNK = 512
TC_STEPS = TC_BLOCKS // TC_CHUNK


def _f1_counts_sc(xview, target):
  mesh = plsc.VectorSubcoreMesh(core_axis_name="c", subcore_axis_name="s")

  @functools.partial(
      pl.kernel,
      mesh=mesh,
      out_type=jax.ShapeDtypeStruct((NUM_WORKERS, LANES), jnp.int32),
      scratch_types=[
          pltpu.VMEM((BLOCKS_PER_WORKER, 2, BLK), jnp.float32),
          pltpu.VMEM((ROWS_PER_WORKER,), jnp.int32),
          pltpu.VMEM((LANES,), jnp.int32),
      ]
      + [pltpu.SemaphoreType.DMA] * (2 * NUM_CHUNKS),
      compiler_params=pltpu.CompilerParams(needs_layout_passes=False),
  )
  def body(x_hbm, t_hbm, out_hbm, x_v, t_v, stage_v, *sems):
    wid = lax.axis_index("s") * 2 + lax.axis_index("c")
    blk0 = wid * BLOCKS_PER_WORKER
    row0 = wid * ROWS_PER_WORKER

    copies = []
    for g in range(NUM_CHUNKS):
      cp_x = pltpu.async_copy(
          x_hbm.at[pl.ds(blk0 + g * BLOCKS_PER_CHUNK, BLOCKS_PER_CHUNK)],
          x_v.at[pl.ds(g * BLOCKS_PER_CHUNK, BLOCKS_PER_CHUNK)],
          sems[2 * g])
      cp_t = pltpu.async_copy(
          t_hbm.at[pl.ds(row0 + g * ROWS_PER_CHUNK, ROWS_PER_CHUNK)],
          t_v.at[pl.ds(g * ROWS_PER_CHUNK, ROWS_PER_CHUNK)],
          sems[2 * g + 1])
      copies.append((cp_x, cp_t))

    lane = lax.iota(jnp.int32, LANES)
    zero = jnp.zeros((LANES,), jnp.int32)
    one = jnp.ones((LANES,), jnp.int32)

    def blk_step(k, carry):
      acc_tp, acc_p, acc_t = carry
      for j in range(VECS_PER_BLK):
        c0 = x_v[k, 0, pl.ds(j * LANES, LANES)]
        c1 = x_v[k, 1, pl.ds(j * LANES, LANES)]
        tv = t_v[pl.ds(k * BLK + j * LANES, LANES)]
        pred = c1 > c0
        acc_p = acc_p + jnp.where(pred, one, zero)
        acc_tp = acc_tp + jnp.where(pred, tv, zero)
        acc_t = acc_t + tv
      return (acc_tp, acc_p, acc_t)

    acc = (zero, zero, zero)
    for g in range(NUM_CHUNKS):
      cp_x, cp_t = copies[g]
      cp_x.wait()
      cp_t.wait()
      acc = lax.fori_loop(
          g * BLOCKS_PER_CHUNK, (g + 1) * BLOCKS_PER_CHUNK, blk_step, acc)
    acc_tp, acc_p, acc_t = acc

    tp_s = jnp.sum(acc_tp)
    p_s = jnp.sum(acc_p)
    t_s = jnp.sum(acc_t)
    stage = jnp.where(lane == 0, tp_s,
                      jnp.where(lane == 1, p_s,
                                jnp.where(lane == 2, t_s, 0)))
    stage_v[...] = stage
    pltpu.sync_copy(stage_v, out_hbm.at[wid])

  return body(xview, target)


def _f1_counts_tc(xview_tc, tview_tc):
  def body(x_ref, t_ref, out_ref):
    i = pl.program_id(0)
    v = x_ref[...]
    q = pltpu.roll(v, 2 * TC_CHUNK - 1, 0)
    row = lax.broadcasted_iota(jnp.int32, (2 * TC_CHUNK, BLK), 0)
    even = (row & 1) == 0
    pred = (q > v) & even
    tv = t_ref[...]
    te = jnp.broadcast_to(tv[:, None, :], (TC_CHUNK, 2, BLK)).reshape(
        2 * TC_CHUNK, BLK)
    zero2 = jnp.zeros((2 * TC_CHUNK, BLK), jnp.int32)
    one2 = jnp.ones((2 * TC_CHUNK, BLK), jnp.int32)
    p_s = jnp.sum(jnp.where(pred, one2, zero2))
    tp_s = jnp.sum(jnp.where(pred, te, zero2))
    t_s = jnp.sum(tv)

    @pl.when(i == 0)
    def _init():
      out_ref[0, 0] = tp_s
      out_ref[0, 1] = p_s
      out_ref[0, 2] = t_s

    @pl.when(i != 0)
    def _acc():
      out_ref[0, 0] += tp_s
      out_ref[0, 1] += p_s
      out_ref[0, 2] += t_s

  return pl.pallas_call(
      body,
      grid=(TC_STEPS,),
      in_specs=[
          pl.BlockSpec((2 * TC_CHUNK, BLK),
                       lambda i: (SC_BLOCKS // TC_CHUNK + i, 0)),
          pl.BlockSpec((TC_CHUNK, BLK),
                       lambda i: (SC_BLOCKS // TC_CHUNK + i, 0)),
      ],
      out_specs=pl.BlockSpec(memory_space=pltpu.SMEM),
      out_shape=jax.ShapeDtypeStruct((1, 3), jnp.int32),
      compiler_params=pltpu.CompilerParams(
          dimension_semantics=("arbitrary",)),
  )(xview_tc, tview_tc)


def _finalize_tc(parts_sc, parts_tc):
  def fin(sc_ref, tc_ref, out_ref):
    x = sc_ref[...]
    col = lax.broadcasted_iota(jnp.int32, (NUM_WORKERS, LANES), 1)
    zero = jnp.zeros((NUM_WORKERS, LANES), jnp.int32)
    tp = (jnp.sum(jnp.where(col == 0, x, zero)) + tc_ref[0, 0]).astype(
        jnp.float32)
    p = (jnp.sum(jnp.where(col == 1, x, zero)) + tc_ref[0, 1]).astype(
        jnp.float32)
    t = (jnp.sum(jnp.where(col == 2, x, zero)) + tc_ref[0, 2]).astype(
        jnp.float32)
    fp = p - tp
    fn = t - tp
    precision = tp / (tp + fp + 1e-10)
    recall = tp / (tp + fn + 1e-10)
    out_ref[0, 0] = 2 * precision * recall / (precision + recall + 1e-10)

  out = pl.pallas_call(
      fin,
      in_specs=[
          pl.BlockSpec(memory_space=pltpu.VMEM),
          pl.BlockSpec(memory_space=pltpu.SMEM),
      ],
      out_specs=pl.BlockSpec(memory_space=pltpu.SMEM),
      out_shape=jax.ShapeDtypeStruct((1, 1), jnp.float32),
  )(parts_sc, parts_tc)
  return out.reshape(())


@jax.jit
def kernel(output, target):
  xview = output.reshape(NUM_BLOCKS, BLK, 2).transpose(0, 2, 1)
  tview = target.reshape(NUM_BLOCKS, BLK)
  parts_sc = _f1_counts_sc(xview, target)
  parts_tc = _f1_counts_tc(xview.reshape(2 * NUM_BLOCKS, BLK), tview)
  return _finalize_tc(parts_sc, parts_tc)
